# baseline (speedup 1.0000x reference)
import jax
import jax.numpy as jnp
from jax.experimental import pallas as pl
from jax.experimental.pallas import tpu as pltpu

_BM = 256


def _fused_kernel(x_ref, w_ref, b_ref, adj_ref, out_ref, h_ref):
    i = pl.program_id(0)
    j = pl.program_id(1)

    @pl.when(jnp.logical_and(i == 0, j == 0))
    def _():
        din = w_ref.shape[1]
        x = x_ref[...]
        h_ref[0] = jnp.dot(x[:, :din], w_ref[0], preferred_element_type=jnp.float32) + b_ref[0]
        h_ref[1] = jnp.dot(x[:, din:], w_ref[1], preferred_element_type=jnp.float32) + b_ref[1]

    part = jnp.dot(adj_ref[0], h_ref[j], preferred_element_type=jnp.float32)

    @pl.when(j == 0)
    def _():
        out_ref[...] = part

    @pl.when(j == 1)
    def _():
        out_ref[...] += part


def kernel(x, adjs, W1, b1, W2, b2):
    n = adjs.shape[1]
    dout = W1.shape[1]
    w = jnp.stack([W1, W2])
    b = jnp.stack([b1, b2]).reshape(2, 1, dout)

    out = pl.pallas_call(
        _fused_kernel,
        grid=(n // _BM, 2),
        in_specs=[
            pl.BlockSpec((n, x.shape[1]), lambda i, j: (0, 0)),
            pl.BlockSpec((2, W1.shape[0], dout), lambda i, j: (0, 0, 0)),
            pl.BlockSpec((2, 1, dout), lambda i, j: (0, 0, 0)),
            pl.BlockSpec((1, _BM, n), lambda i, j: (j, i, 0)),
        ],
        out_specs=pl.BlockSpec((_BM, dout), lambda i, j: (i, 0)),
        out_shape=jax.ShapeDtypeStruct((n, dout), jnp.float32),
        scratch_shapes=[pltpu.VMEM((2, n, dout), jnp.float32)],
        compiler_params=pltpu.CompilerParams(dimension_semantics=("arbitrary", "arbitrary")),
    )(x, w, b, adjs)
    return out


# speedup vs baseline: 1.1941x; 1.1941x over previous
import jax
import jax.numpy as jnp
from jax.experimental import pallas as pl
from jax.experimental.pallas import tpu as pltpu

_BM = 256


def _fused_kernel(x_ref, w_ref, b_ref, adj_ref, out_ref, h_ref):
    @pl.when(pl.program_id(0) == 0)
    def _():
        din = w_ref.shape[1]
        x = x_ref[...]
        h_ref[0] = jnp.dot(x[:, :din], w_ref[0], preferred_element_type=jnp.float32) + b_ref[0]
        h_ref[1] = jnp.dot(x[:, din:], w_ref[1], preferred_element_type=jnp.float32) + b_ref[1]

    out_ref[...] = (
        jnp.dot(adj_ref[0], h_ref[0], preferred_element_type=jnp.float32)
        + jnp.dot(adj_ref[1], h_ref[1], preferred_element_type=jnp.float32)
    )


def kernel(x, adjs, W1, b1, W2, b2):
    n = adjs.shape[1]
    dout = W1.shape[1]
    w = jnp.stack([W1, W2])
    b = jnp.stack([b1, b2]).reshape(2, 1, dout)

    out = pl.pallas_call(
        _fused_kernel,
        grid=(n // _BM,),
        in_specs=[
            pl.BlockSpec((n, x.shape[1]), lambda i: (0, 0)),
            pl.BlockSpec((2, W1.shape[0], dout), lambda i: (0, 0, 0)),
            pl.BlockSpec((2, 1, dout), lambda i: (0, 0, 0)),
            pl.BlockSpec((2, _BM, n), lambda i: (0, i, 0)),
        ],
        out_specs=pl.BlockSpec((_BM, dout), lambda i: (i, 0)),
        out_shape=jax.ShapeDtypeStruct((n, dout), jnp.float32),
        scratch_shapes=[pltpu.VMEM((2, n, dout), jnp.float32)],
        compiler_params=pltpu.CompilerParams(dimension_semantics=("arbitrary",)),
    )(x, w, b, adjs)
    return out


# speedup vs baseline: 1.2059x; 1.0098x over previous
import jax
import jax.numpy as jnp
from jax.experimental import pallas as pl
from jax.experimental.pallas import tpu as pltpu

_BM = 256


def _fused_kernel(x_ref, w_ref, b_ref, adj_ref, out_ref, h_ref):
    @pl.when(pl.program_id(0) == 0)
    def _():
        din = w_ref.shape[1]
        x = x_ref[...]
        h1 = jnp.dot(x[:, :din], w_ref[0], preferred_element_type=jnp.float32) + b_ref[0]
        h2 = jnp.dot(x[:, din:], w_ref[1], preferred_element_type=jnp.float32) + b_ref[1]
        h_ref[0] = h1.astype(jnp.bfloat16)
        h_ref[1] = h2.astype(jnp.bfloat16)

    a1 = adj_ref[0].astype(jnp.bfloat16)
    a2 = adj_ref[1].astype(jnp.bfloat16)
    out_ref[...] = (
        jnp.dot(a1, h_ref[0], preferred_element_type=jnp.float32)
        + jnp.dot(a2, h_ref[1], preferred_element_type=jnp.float32)
    )


def kernel(x, adjs, W1, b1, W2, b2):
    n = adjs.shape[1]
    dout = W1.shape[1]
    w = jnp.stack([W1, W2])
    b = jnp.stack([b1, b2]).reshape(2, 1, dout)

    out = pl.pallas_call(
        _fused_kernel,
        grid=(n // _BM,),
        in_specs=[
            pl.BlockSpec((n, x.shape[1]), lambda i: (0, 0)),
            pl.BlockSpec((2, W1.shape[0], dout), lambda i: (0, 0, 0)),
            pl.BlockSpec((2, 1, dout), lambda i: (0, 0, 0)),
            pl.BlockSpec((2, _BM, n), lambda i: (0, i, 0)),
        ],
        out_specs=pl.BlockSpec((_BM, dout), lambda i: (i, 0)),
        out_shape=jax.ShapeDtypeStruct((n, dout), jnp.float32),
        scratch_shapes=[pltpu.VMEM((2, n, dout), jnp.bfloat16)],
        compiler_params=pltpu.CompilerParams(dimension_semantics=("arbitrary",)),
    )(x, w, b, adjs)
    return out
